# baseline (speedup 1.0000x reference)
import functools
import math

import jax
import jax.numpy as jnp
from jax import lax
from jax.experimental import pallas as pl
from jax.experimental.pallas import tpu as pltpu
from jax.experimental.pallas import tpu_sc as plsc

D_MODEL = 1024
SCALE = math.sqrt(D_MODEL)
LANES = 16


@functools.lru_cache(maxsize=None)
def _build(n_idx: int, vocab: int, d: int):
    info = plsc.get_sparse_core_info()
    nc, ns = info.num_cores, info.num_subcores
    nw = nc * ns
    assert n_idx % nw == 0
    per_w = n_idx // nw
    chunk = 64
    assert per_w % chunk == 0
    n_chunks = per_w // chunk
    vregs_per_row = d // LANES

    mesh = plsc.VectorSubcoreMesh(core_axis_name="c", subcore_axis_name="s")

    @functools.partial(
        pl.kernel,
        mesh=mesh,
        out_type=jax.ShapeDtypeStruct((n_idx, d), jnp.float32),
        scratch_types=[
            pltpu.VMEM((per_w,), jnp.int32),
            pltpu.VMEM((chunk, d), jnp.float32),
            pltpu.SemaphoreType.DMA,
        ],
    )
    def emb(x_hbm, table_hbm, out_hbm, idx_v, rows_v, sem):
        wid = lax.axis_index("s") * nc + lax.axis_index("c")
        base = wid * per_w
        pltpu.sync_copy(x_hbm.at[pl.ds(base, per_w)], idx_v)

        def chunk_body(c, carry):
            pltpu.async_copy(
                table_hbm.at[idx_v.at[pl.ds(c * chunk, chunk)]], rows_v, sem
            ).wait()

            def row_body(r, carry2):
                def vec_body(j, carry3):
                    sl = pl.ds(j * LANES, LANES)
                    rows_v[r, sl] = rows_v[r, sl] * SCALE
                    return carry3

                return lax.fori_loop(0, vregs_per_row, vec_body, carry2)

            lax.fori_loop(0, chunk, row_body, 0)
            pltpu.sync_copy(rows_v, out_hbm.at[pl.ds(base + c * chunk, chunk)])
            return carry

        lax.fori_loop(0, n_chunks, chunk_body, 0)

    return emb


def kernel(x, table):
    b, s = x.shape
    vocab, d = table.shape
    flat = x.reshape(b * s).astype(jnp.int32)
    out = _build(b * s, vocab, d)(flat, table)
    return out.reshape(b, s, d)


# speedup vs baseline: 2.5726x; 2.5726x over previous
import functools
import math

import jax
import jax.numpy as jnp
from jax import lax
from jax.experimental import pallas as pl
from jax.experimental.pallas import tpu as pltpu
from jax.experimental.pallas import tpu_sc as plsc

D_MODEL = 1024
SCALE = math.sqrt(D_MODEL)
LANES = 16


@functools.lru_cache(maxsize=None)
def _build(n_idx: int, vocab: int, d: int):
    info = plsc.get_sparse_core_info()
    nc, ns = info.num_cores, info.num_subcores
    nw = nc * ns
    assert n_idx % nw == 0
    per_w = n_idx // nw
    chunk = 32
    assert per_w % chunk == 0
    n_chunks = per_w // chunk
    vregs_per_row = d // LANES
    nbuf = 3

    mesh = plsc.VectorSubcoreMesh(core_axis_name="c", subcore_axis_name="s")

    @functools.partial(
        pl.kernel,
        mesh=mesh,
        out_type=jax.ShapeDtypeStruct((n_idx, d), jnp.float32),
        scratch_types=[
            pltpu.VMEM((per_w,), jnp.int32),
            pltpu.VMEM((nbuf, chunk, d), jnp.float32),
            pltpu.SemaphoreType.DMA((nbuf,)),
            pltpu.SemaphoreType.DMA((nbuf,)),
        ],
    )
    def emb(x_hbm, table_hbm, out_hbm, idx_v, rows_v, gsem, wsem):
        wid = lax.axis_index("s") * nc + lax.axis_index("c")
        base = wid * per_w
        pltpu.sync_copy(x_hbm.at[pl.ds(base, per_w)], idx_v)

        def gather(c):
            return pltpu.async_copy(
                table_hbm.at[idx_v.at[pl.ds(c * chunk, chunk)]],
                rows_v.at[c % nbuf],
                gsem.at[c % nbuf],
            )

        def write(c):
            return pltpu.async_copy(
                rows_v.at[c % nbuf],
                out_hbm.at[pl.ds(base + c * chunk, chunk)],
                wsem.at[c % nbuf],
            )

        def scale(c):
            buf = rows_v.at[c % nbuf]

            def row_body(r, carry):
                for j in range(vregs_per_row):
                    sl = pl.ds(j * LANES, LANES)
                    buf[r, sl] = buf[r, sl] * SCALE
                return carry

            lax.fori_loop(0, chunk, row_body, 0)

        g = [None] * n_chunks
        w = [None] * n_chunks
        g[0] = gather(0)
        if n_chunks > 1:
            g[1] = gather(1)
        for c in range(n_chunks):
            g[c].wait()
            scale(c)
            w[c] = write(c)
            if c + 2 < n_chunks:
                if c >= 1:
                    w[c - 1].wait()
                g[c + 2] = gather(c + 2)
        for c in range(max(0, n_chunks - 3), n_chunks):
            w[c].wait()

    return emb


def kernel(x, table):
    b, s = x.shape
    vocab, d = table.shape
    flat = x.reshape(b * s).astype(jnp.int32)
    out = _build(b * s, vocab, d)(flat, table)
    return out.reshape(b, s, d)
